# baseline (speedup 1.0000x reference)
import functools

import jax
import jax.numpy as jnp
from jax import lax
from jax.experimental import pallas as pl
from jax.experimental.pallas import tpu as pltpu
from jax.experimental.pallas import tpu_sc as plsc


def _argmax_body(z_ref, sel_ref):
    z = z_ref[...]
    b, k = z.shape
    m = jnp.max(z, axis=1, keepdims=True)
    iota = lax.broadcasted_iota(jnp.int32, (b, k), 1)
    cand = jnp.where(z == m, iota, k)
    sel = jnp.min(cand, axis=1, keepdims=True)
    onehot = (iota == sel).astype(jnp.float32)
    iota_row = lax.broadcasted_iota(jnp.int32, (1, k), 1).astype(jnp.float32)
    selrow = lax.dot_general(iota_row, onehot, (((1,), (1,)), ((), ())),
                             preferred_element_type=jnp.float32)
    sel_ref[...] = selrow.astype(jnp.int32).reshape(1, 1, b)


def _argmax_tc(z, block_rows=2048):
    n, k = z.shape
    g = n // block_rows
    return pl.pallas_call(
        _argmax_body,
        grid=(g,),
        in_specs=[pl.BlockSpec((block_rows, k), lambda j: (j, 0))],
        out_specs=pl.BlockSpec((1, 1, block_rows), lambda j: (j, 0, 0)),
        out_shape=jax.ShapeDtypeStruct((g, 1, block_rows), jnp.int32),
    )(z)


def _gather_sc(table, idx1d, n, k):
    info = plsc.get_sparse_core_info()
    nc, ns = info.num_cores, info.num_subcores
    nw = nc * ns
    rows_per_w = n // nw
    chunk = 256
    n_chunks = rows_per_w // chunk
    gpc = chunk // 128
    mesh = plsc.VectorSubcoreMesh(core_axis_name="c", subcore_axis_name="s")

    @functools.partial(
        pl.kernel,
        mesh=mesh,
        out_type=jax.ShapeDtypeStruct((n, k), jnp.float32),
        scratch_types=[
            pltpu.VMEM((rows_per_w,), jnp.int32),
            pltpu.VMEM((2, chunk, k), jnp.float32),
            pltpu.SemaphoreType.DMA,
            pltpu.SemaphoreType.DMA,
            pltpu.SemaphoreType.DMA,
        ],
    )
    def gather_kernel(table_hbm, idx_hbm, out_hbm, idx_v, rows_v, gsem,
                      osem0, osem1):
        wid = lax.axis_index("s") * nc + lax.axis_index("c")
        row0 = wid * rows_per_w
        pltpu.sync_copy(idx_hbm.at[pl.ds(row0, rows_per_w)], idx_v)
        osems = (osem0, osem1)

        def do_chunk(c, slot):
            off = row0 + c * chunk
            @pl.when(c >= 2)
            def _():
                pltpu.make_async_copy(
                    rows_v.at[slot],
                    out_hbm.at[pl.ds(off - 2 * chunk, chunk)],
                    osems[slot],
                ).wait()
            handles = []
            for j in range(gpc):
                handles.append(pltpu.async_copy(
                    table_hbm.at[idx_v.at[pl.ds(c * chunk + j * 128, 128)]],
                    rows_v.at[slot].at[pl.ds(j * 128, 128)],
                    gsem,
                ))
            for h in handles:
                h.wait()
            pltpu.async_copy(rows_v.at[slot],
                             out_hbm.at[pl.ds(off, chunk)], osems[slot])

        def body(i2, carry):
            for b in range(2):
                do_chunk(i2 * 2 + b, b)
            return carry

        lax.fori_loop(0, n_chunks // 2, body, 0)
        for b in range(2):
            c = n_chunks - 2 + b
            pltpu.make_async_copy(
                rows_v.at[b],
                out_hbm.at[pl.ds(row0 + c * chunk, chunk)],
                osems[b],
            ).wait()

    return gather_kernel(table, idx1d)


def kernel(z, A, z0):
    n, k = z.shape
    sel = _argmax_tc(z).reshape(n)
    idx = jnp.concatenate([jnp.full((1,), k, jnp.int32), sel[:-1]])
    table = jnp.concatenate([A, z0.astype(A.dtype)], axis=0)
    return _gather_sc(table, idx, n, k)


# speedup vs baseline: 1.1743x; 1.1743x over previous
import functools

import jax
import jax.numpy as jnp
from jax import lax
from jax.experimental import pallas as pl
from jax.experimental.pallas import tpu as pltpu
from jax.experimental.pallas import tpu_sc as plsc


def _argmax_body(z_ref, sel_ref):
    z = z_ref[...]
    b, k = z.shape
    m = jnp.max(z, axis=1, keepdims=True)
    eq = (z == m).astype(jnp.float32)
    iota_r = lax.broadcasted_iota(jnp.int32, (k, k), 0)
    iota_c = lax.broadcasted_iota(jnp.int32, (k, k), 1)
    ut = (iota_r < iota_c).astype(jnp.float32)
    s = lax.dot_general(eq, ut, (((1,), (0,)), ((), ())),
                        preferred_element_type=jnp.float32)
    first = eq * (s == 0)
    iota_row = lax.broadcasted_iota(jnp.int32, (1, k), 1).astype(jnp.float32)
    selrow = lax.dot_general(iota_row, first, (((1,), (1,)), ((), ())),
                             preferred_element_type=jnp.float32)
    sel_ref[...] = selrow.astype(jnp.int32).reshape(1, 1, b)


def _argmax_tc(z, block_rows=4096):
    n, k = z.shape
    g = n // block_rows
    return pl.pallas_call(
        _argmax_body,
        grid=(g,),
        in_specs=[pl.BlockSpec((block_rows, k), lambda j: (j, 0))],
        out_specs=pl.BlockSpec((1, 1, block_rows), lambda j: (j, 0, 0)),
        out_shape=jax.ShapeDtypeStruct((g, 1, block_rows), jnp.int32),
    )(z)


def _gather_sc(table, idx1d, n, k):
    info = plsc.get_sparse_core_info()
    nc, ns = info.num_cores, info.num_subcores
    nw = nc * ns
    rows_per_w = n // nw
    chunk = 512
    n_chunks = rows_per_w // chunk
    gpc = chunk // 128
    mesh = plsc.VectorSubcoreMesh(core_axis_name="c", subcore_axis_name="s")

    @functools.partial(
        pl.kernel,
        mesh=mesh,
        out_type=jax.ShapeDtypeStruct((n, k), jnp.float32),
        scratch_types=[
            pltpu.VMEM((rows_per_w,), jnp.int32),
            pltpu.VMEM((chunk, k), jnp.float32),
            pltpu.SemaphoreType.DMA,
        ],
    )
    def gather_kernel(table_hbm, idx_hbm, out_hbm, idx_v, rows_v, gsem):
        wid = lax.axis_index("s") * nc + lax.axis_index("c")
        row0 = wid * rows_per_w
        pltpu.sync_copy(idx_hbm.at[pl.ds(row0, rows_per_w)], idx_v)

        def body(c, carry):
            off = row0 + c * chunk
            handles = []
            for j in range(gpc):
                handles.append(pltpu.async_copy(
                    table_hbm.at[idx_v.at[pl.ds(c * chunk + j * 128, 128)]],
                    rows_v.at[pl.ds(j * 128, 128)],
                    gsem,
                ))
            for h in handles:
                h.wait()
            pltpu.sync_copy(rows_v, out_hbm.at[pl.ds(off, chunk)])
            return carry

        lax.fori_loop(0, n_chunks, body, 0)

    return gather_kernel(table, idx1d)


def kernel(z, A, z0):
    n, k = z.shape
    sel = _argmax_tc(z).reshape(n)
    idx = jnp.concatenate([jnp.full((1,), k, jnp.int32), sel[:-1]])
    table = jnp.concatenate([A, z0.astype(A.dtype)], axis=0)
    return _gather_sc(table, idx, n, k)


# speedup vs baseline: 2.6817x; 2.2836x over previous
import functools

import jax
import jax.numpy as jnp
from jax import lax
from jax.experimental import pallas as pl
from jax.experimental.pallas import tpu as pltpu
from jax.experimental.pallas import tpu_sc as plsc


def _argmax_body(z_ref, sel_ref):
    z = z_ref[...]
    b, k = z.shape
    m = jnp.max(z, axis=1, keepdims=True)
    eq = (z == m).astype(jnp.float32)
    iota_r = lax.broadcasted_iota(jnp.int32, (k, k), 0)
    iota_c = lax.broadcasted_iota(jnp.int32, (k, k), 1)
    ut = (iota_r < iota_c).astype(jnp.float32)
    s = lax.dot_general(eq, ut, (((1,), (0,)), ((), ())),
                        preferred_element_type=jnp.float32)
    first = eq * (s == 0)
    iota_row = lax.broadcasted_iota(jnp.int32, (1, k), 1).astype(jnp.float32)
    selrow = lax.dot_general(iota_row, first, (((1,), (1,)), ((), ())),
                             preferred_element_type=jnp.float32)
    sel_ref[...] = selrow.astype(jnp.int32).reshape(1, 1, b)


def _argmax_tc(z, block_rows=4096):
    n, k = z.shape
    g = n // block_rows
    return pl.pallas_call(
        _argmax_body,
        grid=(g,),
        in_specs=[pl.BlockSpec((block_rows, k), lambda j: (j, 0))],
        out_specs=pl.BlockSpec((1, 1, block_rows), lambda j: (j, 0, 0)),
        out_shape=jax.ShapeDtypeStruct((g, 1, block_rows), jnp.int32),
    )(z)


def _gather_sc(table, idx1d, n, k):
    info = plsc.get_sparse_core_info()
    nc, ns = info.num_cores, info.num_subcores
    nw = nc * ns
    rows_per_w = n // nw
    chunk = 512
    n_chunks = rows_per_w // chunk
    gpc = chunk // 128
    mesh = plsc.VectorSubcoreMesh(core_axis_name="c", subcore_axis_name="s")

    @functools.partial(
        pl.kernel,
        mesh=mesh,
        out_type=jax.ShapeDtypeStruct((n, k), jnp.float32),
        scratch_types=[
            pltpu.VMEM((rows_per_w,), jnp.int32),
            pltpu.VMEM((chunk, k), jnp.float32),
            pltpu.VMEM_SHARED((k + 1, k), jnp.float32),
            pltpu.SemaphoreType.DMA,
        ],
    )
    def gather_kernel(table_hbm, idx_hbm, out_hbm, idx_v, rows_v, table_spm,
                      gsem):
        wid = lax.axis_index("s") * nc + lax.axis_index("c")
        row0 = wid * rows_per_w

        @pl.when(lax.axis_index("s") == 0)
        def _():
            pltpu.sync_copy(table_hbm, table_spm)
        pltpu.sync_copy(idx_hbm.at[pl.ds(row0, rows_per_w)], idx_v)
        plsc.subcore_barrier()

        def body(c, carry):
            off = row0 + c * chunk
            handles = []
            for j in range(gpc):
                handles.append(pltpu.async_copy(
                    table_spm.at[idx_v.at[pl.ds(c * chunk + j * 128, 128)]],
                    rows_v.at[pl.ds(j * 128, 128)],
                    gsem,
                ))
            for h in handles:
                h.wait()
            pltpu.sync_copy(rows_v, out_hbm.at[pl.ds(off, chunk)])
            return carry

        lax.fori_loop(0, n_chunks, body, 0)

    return gather_kernel(table, idx1d)


def kernel(z, A, z0):
    n, k = z.shape
    sel = _argmax_tc(z).reshape(n)
    idx = jnp.concatenate([jnp.full((1,), k, jnp.int32), sel[:-1]])
    table = jnp.concatenate([A, z0.astype(A.dtype)], axis=0)
    return _gather_sc(table, idx, n, k)


# speedup vs baseline: 3.6400x; 1.3574x over previous
import functools

import jax
import jax.numpy as jnp
from jax import lax
from jax.experimental import pallas as pl
from jax.experimental.pallas import tpu as pltpu
from jax.experimental.pallas import tpu_sc as plsc


def _argmax_body(z_ref, sel_ref):
    z = z_ref[...]
    b, k = z.shape
    m = jnp.max(z, axis=1, keepdims=True)
    eq = (z == m).astype(jnp.float32)
    iota_r = lax.broadcasted_iota(jnp.int32, (k, k), 0)
    iota_c = lax.broadcasted_iota(jnp.int32, (k, k), 1)
    ut = (iota_r < iota_c).astype(jnp.float32)
    s = lax.dot_general(eq, ut, (((1,), (0,)), ((), ())),
                        preferred_element_type=jnp.float32)
    first = eq * (s == 0)
    iota_row = lax.broadcasted_iota(jnp.int32, (1, k), 1).astype(jnp.float32)
    selrow = lax.dot_general(iota_row, first, (((1,), (1,)), ((), ())),
                             preferred_element_type=jnp.float32)
    sel_ref[...] = selrow.astype(jnp.int32).reshape(1, 1, b)


def _argmax_tc(z, block_rows=8192):
    n, k = z.shape
    g = n // block_rows
    return pl.pallas_call(
        _argmax_body,
        grid=(g,),
        in_specs=[pl.BlockSpec((block_rows, k), lambda j: (j, 0))],
        out_specs=pl.BlockSpec((1, 1, block_rows), lambda j: (j, 0, 0)),
        out_shape=jax.ShapeDtypeStruct((g, 1, block_rows), jnp.int32),
    )(z)


def _gather_sc(table, idx1d, n, k):
    info = plsc.get_sparse_core_info()
    nc, ns = info.num_cores, info.num_subcores
    nw = nc * ns
    rows_per_w = n // nw
    chunk = 256
    n_chunks = rows_per_w // chunk
    gpc = chunk // 128
    mesh = plsc.VectorSubcoreMesh(core_axis_name="c", subcore_axis_name="s")

    @functools.partial(
        pl.kernel,
        mesh=mesh,
        out_type=jax.ShapeDtypeStruct((n, k), jnp.float32),
        scratch_types=[
            pltpu.VMEM((rows_per_w,), jnp.int32),
            pltpu.VMEM((2, chunk, k), jnp.float32),
            pltpu.VMEM_SHARED((k + 1, k), jnp.float32),
            pltpu.SemaphoreType.DMA,
            pltpu.SemaphoreType.DMA,
            pltpu.SemaphoreType.DMA,
        ],
    )
    def gather_kernel(table_hbm, idx_hbm, out_hbm, idx_v, rows_v, table_spm,
                      gsem, osem0, osem1):
        wid = lax.axis_index("s") * nc + lax.axis_index("c")
        row0 = wid * rows_per_w

        @pl.when(lax.axis_index("s") == 0)
        def _():
            pltpu.sync_copy(table_hbm, table_spm)
        pltpu.sync_copy(idx_hbm.at[pl.ds(row0, rows_per_w)], idx_v)
        plsc.subcore_barrier()
        osems = (osem0, osem1)

        def do_chunk(c, slot):
            off = row0 + c * chunk
            @pl.when(c >= 2)
            def _():
                pltpu.make_async_copy(
                    rows_v.at[slot],
                    out_hbm.at[pl.ds(off - 2 * chunk, chunk)],
                    osems[slot],
                ).wait()
            handles = []
            for j in range(gpc):
                handles.append(pltpu.async_copy(
                    table_spm.at[idx_v.at[pl.ds(c * chunk + j * 128, 128)]],
                    rows_v.at[slot].at[pl.ds(j * 128, 128)],
                    gsem,
                ))
            for h in handles:
                h.wait()
            pltpu.async_copy(rows_v.at[slot],
                             out_hbm.at[pl.ds(off, chunk)], osems[slot])

        def body(i2, carry):
            for b in range(2):
                do_chunk(i2 * 2 + b, b)
            return carry

        lax.fori_loop(0, n_chunks // 2, body, 0)
        for b in range(2):
            c = n_chunks - 2 + b
            pltpu.make_async_copy(
                rows_v.at[b],
                out_hbm.at[pl.ds(row0 + c * chunk, chunk)],
                osems[b],
            ).wait()

    return gather_kernel(table, idx1d)


def kernel(z, A, z0):
    n, k = z.shape
    sel = _argmax_tc(z).reshape(n)
    idx = jnp.concatenate([jnp.full((1,), k, jnp.int32), sel[:-1]])
    table = jnp.concatenate([A, z0.astype(A.dtype)], axis=0)
    return _gather_sc(table, idx, n, k)


# speedup vs baseline: 3.9175x; 1.0763x over previous
import functools

import jax
import jax.numpy as jnp
from jax import lax
from jax.experimental import pallas as pl
from jax.experimental.pallas import tpu as pltpu
from jax.experimental.pallas import tpu_sc as plsc


def _argmax_body(z_ref, sel_ref):
    z = z_ref[...]
    b, k = z.shape
    m = jnp.max(z, axis=1, keepdims=True)
    eq = (z == m).astype(jnp.float32)
    iota_r = lax.broadcasted_iota(jnp.int32, (k, k), 0)
    iota_c = lax.broadcasted_iota(jnp.int32, (k, k), 1)
    ut = (iota_r < iota_c).astype(jnp.float32)
    s = lax.dot_general(eq, ut, (((1,), (0,)), ((), ())),
                        preferred_element_type=jnp.float32)
    first = eq * (s == 0)
    iota_row = lax.broadcasted_iota(jnp.int32, (1, k), 1).astype(jnp.float32)
    selrow = lax.dot_general(iota_row, first, (((1,), (1,)), ((), ())),
                             preferred_element_type=jnp.float32)
    sel_ref[...] = selrow.astype(jnp.int32).reshape(1, 1, b)


def _argmax_tc(z, block_rows=16384):
    n, k = z.shape
    g = n // block_rows
    return pl.pallas_call(
        _argmax_body,
        grid=(g,),
        in_specs=[pl.BlockSpec((block_rows, k), lambda j: (j, 0))],
        out_specs=pl.BlockSpec((1, 1, block_rows), lambda j: (j, 0, 0)),
        out_shape=jax.ShapeDtypeStruct((g, 1, block_rows), jnp.int32),
    )(z)


def _gather_sc(table, idx1d, n, k):
    info = plsc.get_sparse_core_info()
    nc, ns = info.num_cores, info.num_subcores
    nw = nc * ns
    rows_per_w = n // nw
    chunk = 256
    n_chunks = rows_per_w // chunk
    gpc = chunk // 128
    mesh = plsc.VectorSubcoreMesh(core_axis_name="c", subcore_axis_name="s")

    @functools.partial(
        pl.kernel,
        mesh=mesh,
        out_type=jax.ShapeDtypeStruct((n, k), jnp.float32),
        scratch_types=[
            pltpu.VMEM((rows_per_w,), jnp.int32),
            pltpu.VMEM((2, chunk, k), jnp.float32),
            pltpu.VMEM_SHARED((k + 1, k), jnp.float32),
            pltpu.SemaphoreType.DMA,
            pltpu.SemaphoreType.DMA,
            pltpu.SemaphoreType.DMA,
        ],
    )
    def gather_kernel(table_hbm, idx_hbm, out_hbm, idx_v, rows_v, table_spm,
                      gsem, osem0, osem1):
        wid = lax.axis_index("s") * nc + lax.axis_index("c")
        row0 = wid * rows_per_w

        @pl.when(lax.axis_index("s") == 0)
        def _():
            pltpu.sync_copy(table_hbm, table_spm)
        pltpu.sync_copy(idx_hbm.at[pl.ds(row0, rows_per_w)], idx_v)
        plsc.subcore_barrier()
        osems = (osem0, osem1)

        def do_chunk(c, slot):
            off = row0 + c * chunk
            @pl.when(c >= 2)
            def _():
                pltpu.make_async_copy(
                    rows_v.at[slot],
                    out_hbm.at[pl.ds(off - 2 * chunk, chunk)],
                    osems[slot],
                ).wait()
            handles = []
            for j in range(gpc):
                handles.append(pltpu.async_copy(
                    table_spm.at[idx_v.at[pl.ds(c * chunk + j * 128, 128)]],
                    rows_v.at[slot].at[pl.ds(j * 128, 128)],
                    gsem,
                ))
            for h in handles:
                h.wait()
            pltpu.async_copy(rows_v.at[slot],
                             out_hbm.at[pl.ds(off, chunk)], osems[slot])

        def body(i2, carry):
            for b in range(2):
                do_chunk(i2 * 2 + b, b)
            return carry

        lax.fori_loop(0, n_chunks // 2, body, 0)
        for b in range(2):
            c = n_chunks - 2 + b
            pltpu.make_async_copy(
                rows_v.at[b],
                out_hbm.at[pl.ds(row0 + c * chunk, chunk)],
                osems[b],
            ).wait()

    return gather_kernel(table, idx1d)


def kernel(z, A, z0):
    n, k = z.shape
    sel = _argmax_tc(z).reshape(n)
    idx = jnp.concatenate([jnp.full((1,), k, jnp.int32), sel[:-1]])
    table = jnp.concatenate([A, z0.astype(A.dtype)], axis=0)
    return _gather_sc(table, idx, n, k)


# speedup vs baseline: 4.0506x; 1.0340x over previous
import functools

import jax
import jax.numpy as jnp
from jax import lax
from jax.experimental import pallas as pl
from jax.experimental.pallas import tpu as pltpu
from jax.experimental.pallas import tpu_sc as plsc


def _argmax_body(z_ref, sel_ref):
    z = z_ref[...]
    b, k = z.shape
    m = jnp.max(z, axis=1, keepdims=True)
    eq = (z == m).astype(jnp.float32)
    iota_r = lax.broadcasted_iota(jnp.int32, (k, k), 0)
    iota_c = lax.broadcasted_iota(jnp.int32, (k, k), 1)
    ut = (iota_r < iota_c).astype(jnp.float32)
    s = lax.dot_general(eq, ut, (((1,), (0,)), ((), ())),
                        preferred_element_type=jnp.float32)
    first = eq * (s == 0)
    iota_row = lax.broadcasted_iota(jnp.int32, (1, k), 1).astype(jnp.float32)
    selrow = lax.dot_general(iota_row, first, (((1,), (1,)), ((), ())),
                             preferred_element_type=jnp.float32)
    sel_ref[...] = selrow.astype(jnp.int32).reshape(1, 1, b)


def _argmax_tc(z, block_rows=32768):
    n, k = z.shape
    g = n // block_rows
    return pl.pallas_call(
        _argmax_body,
        grid=(g,),
        in_specs=[pl.BlockSpec((block_rows, k), lambda j: (j, 0))],
        out_specs=pl.BlockSpec((1, 1, block_rows), lambda j: (j, 0, 0)),
        out_shape=jax.ShapeDtypeStruct((g, 1, block_rows), jnp.int32),
    )(z)


def _gather_sc(table, idx1d, n, k):
    info = plsc.get_sparse_core_info()
    nc, ns = info.num_cores, info.num_subcores
    nw = nc * ns
    rows_per_w = n // nw
    chunk = 256
    n_chunks = rows_per_w // chunk
    gpc = chunk // 128
    mesh = plsc.VectorSubcoreMesh(core_axis_name="c", subcore_axis_name="s")

    @functools.partial(
        pl.kernel,
        mesh=mesh,
        out_type=jax.ShapeDtypeStruct((n, k), jnp.float32),
        scratch_types=[
            pltpu.VMEM((rows_per_w,), jnp.int32),
            pltpu.VMEM((2, chunk, k), jnp.float32),
            pltpu.VMEM_SHARED((k + 1, k), jnp.float32),
            pltpu.SemaphoreType.DMA,
            pltpu.SemaphoreType.DMA,
            pltpu.SemaphoreType.DMA,
        ],
    )
    def gather_kernel(table_hbm, idx_hbm, out_hbm, idx_v, rows_v, table_spm,
                      gsem, osem0, osem1):
        wid = lax.axis_index("s") * nc + lax.axis_index("c")
        row0 = wid * rows_per_w

        @pl.when(lax.axis_index("s") == 0)
        def _():
            pltpu.sync_copy(table_hbm, table_spm)
        pltpu.sync_copy(idx_hbm.at[pl.ds(row0, rows_per_w)], idx_v)
        plsc.subcore_barrier()
        osems = (osem0, osem1)

        def do_chunk(c, slot):
            off = row0 + c * chunk
            @pl.when(c >= 2)
            def _():
                pltpu.make_async_copy(
                    rows_v.at[slot],
                    out_hbm.at[pl.ds(off - 2 * chunk, chunk)],
                    osems[slot],
                ).wait()
            handles = []
            for j in range(gpc):
                handles.append(pltpu.async_copy(
                    table_spm.at[idx_v.at[pl.ds(c * chunk + j * 128, 128)]],
                    rows_v.at[slot].at[pl.ds(j * 128, 128)],
                    gsem,
                ))
            for h in handles:
                h.wait()
            pltpu.async_copy(rows_v.at[slot],
                             out_hbm.at[pl.ds(off, chunk)], osems[slot])

        def body(i2, carry):
            for b in range(2):
                do_chunk(i2 * 2 + b, b)
            return carry

        lax.fori_loop(0, n_chunks // 2, body, 0)
        for b in range(2):
            c = n_chunks - 2 + b
            pltpu.make_async_copy(
                rows_v.at[b],
                out_hbm.at[pl.ds(row0 + c * chunk, chunk)],
                osems[b],
            ).wait()

    return gather_kernel(table, idx1d)


def kernel(z, A, z0):
    n, k = z.shape
    sel = _argmax_tc(z).reshape(n)
    idx = jnp.concatenate([jnp.full((1,), k, jnp.int32), sel[:-1]])
    table = jnp.concatenate([A, z0.astype(A.dtype)], axis=0)
    return _gather_sc(table, idx, n, k)


# speedup vs baseline: 4.0516x; 1.0003x over previous
import functools

import jax
import jax.numpy as jnp
from jax import lax
from jax.experimental import pallas as pl
from jax.experimental.pallas import tpu as pltpu
from jax.experimental.pallas import tpu_sc as plsc


def _argmax_body(z_ref, sel_ref):
    z = z_ref[...]
    b, k = z.shape
    m = jnp.max(z, axis=1, keepdims=True)
    eq = (z == m).astype(jnp.float32)
    iota_r = lax.broadcasted_iota(jnp.int32, (k, k), 0)
    iota_c = lax.broadcasted_iota(jnp.int32, (k, k), 1)
    ut = (iota_r < iota_c).astype(jnp.float32)
    s = lax.dot_general(eq, ut, (((1,), (0,)), ((), ())),
                        preferred_element_type=jnp.float32)
    first = eq * (s == 0)
    iota_row = lax.broadcasted_iota(jnp.int32, (1, k), 1).astype(jnp.float32)
    selrow = lax.dot_general(iota_row, first, (((1,), (1,)), ((), ())),
                             preferred_element_type=jnp.float32)
    sel_ref[...] = selrow.astype(jnp.int32).reshape(1, 1, b)


def _argmax_tc(z, block_rows=32768):
    n, k = z.shape
    g = n // block_rows
    return pl.pallas_call(
        _argmax_body,
        grid=(g,),
        in_specs=[pl.BlockSpec((block_rows, k), lambda j: (j, 0))],
        out_specs=pl.BlockSpec((1, 1, block_rows), lambda j: (j, 0, 0)),
        out_shape=jax.ShapeDtypeStruct((g, 1, block_rows), jnp.int32),
    )(z)


def _gather_sc(table, idx1d, n, k):
    info = plsc.get_sparse_core_info()
    nc, ns = info.num_cores, info.num_subcores
    nw = nc * ns
    rows_per_w = n // nw
    chunk = 256
    n_chunks = rows_per_w // chunk
    gpc = chunk // 128
    mesh = plsc.VectorSubcoreMesh(core_axis_name="c", subcore_axis_name="s")

    nbuf = 3
    scratch = [
        pltpu.VMEM((rows_per_w,), jnp.int32),
        pltpu.VMEM((nbuf, chunk, k), jnp.float32),
        pltpu.VMEM_SHARED((k + 1, k), jnp.float32),
        pltpu.SemaphoreType.DMA,
    ] + [pltpu.SemaphoreType.DMA] * nbuf

    @functools.partial(
        pl.kernel,
        mesh=mesh,
        out_type=jax.ShapeDtypeStruct((n, k), jnp.float32),
        scratch_types=scratch,
    )
    def gather_kernel(table_hbm, idx_hbm, out_hbm, idx_v, rows_v, table_spm,
                      gsem, *osems):
        wid = lax.axis_index("s") * nc + lax.axis_index("c")
        row0 = wid * rows_per_w

        @pl.when(lax.axis_index("s") == 0)
        def _():
            pltpu.sync_copy(table_hbm, table_spm)
        pltpu.sync_copy(idx_hbm.at[pl.ds(row0, rows_per_w)], idx_v)
        plsc.subcore_barrier()

        def do_chunk(c, slot):
            off = row0 + c * chunk
            @pl.when(c >= nbuf)
            def _():
                pltpu.make_async_copy(
                    rows_v.at[slot],
                    out_hbm.at[pl.ds(off - nbuf * chunk, chunk)],
                    osems[slot],
                ).wait()
            handles = []
            for j in range(gpc):
                handles.append(pltpu.async_copy(
                    table_spm.at[idx_v.at[pl.ds(c * chunk + j * 128, 128)]],
                    rows_v.at[slot].at[pl.ds(j * 128, 128)],
                    gsem,
                ))
            for h in handles:
                h.wait()
            pltpu.async_copy(rows_v.at[slot],
                             out_hbm.at[pl.ds(off, chunk)], osems[slot])

        n_main = (n_chunks // nbuf) * nbuf

        def body(i2, carry):
            for b in range(nbuf):
                do_chunk(i2 * nbuf + b, b)
            return carry

        lax.fori_loop(0, n_chunks // nbuf, body, 0)
        for c in range(n_main, n_chunks):
            do_chunk(c, c % nbuf)
        for c in range(n_chunks - nbuf, n_chunks):
            slot = c % nbuf
            pltpu.make_async_copy(
                rows_v.at[slot],
                out_hbm.at[pl.ds(row0 + c * chunk, chunk)],
                osems[slot],
            ).wait()

    return gather_kernel(table, idx1d)


def kernel(z, A, z0):
    n, k = z.shape
    sel = _argmax_tc(z).reshape(n)
    idx = jnp.concatenate([jnp.full((1,), k, jnp.int32), sel[:-1]])
    table = jnp.concatenate([A, z0.astype(A.dtype)], axis=0)
    return _gather_sc(table, idx, n, k)


# speedup vs baseline: 4.0681x; 1.0041x over previous
import functools

import jax
import jax.numpy as jnp
from jax import lax
from jax.experimental import pallas as pl
from jax.experimental.pallas import tpu as pltpu
from jax.experimental.pallas import tpu_sc as plsc


def _argmax_body(z_ref, sel_ref):
    z = z_ref[...]
    b, k = z.shape
    m = jnp.max(z, axis=1, keepdims=True)
    eq = (z == m).astype(jnp.float32)
    iota_r = lax.broadcasted_iota(jnp.int32, (k, k), 0)
    iota_c = lax.broadcasted_iota(jnp.int32, (k, k), 1)
    ut = (iota_r <= iota_c).astype(jnp.float32)
    s = lax.dot_general(eq, ut, (((1,), (0,)), ((), ())),
                        preferred_element_type=jnp.float32)
    zeros = (s == 0).astype(jnp.float32)
    ones_row = jnp.ones((1, k), jnp.float32)
    selrow = lax.dot_general(ones_row, zeros, (((1,), (1,)), ((), ())),
                             preferred_element_type=jnp.float32)
    sel_ref[...] = selrow.astype(jnp.int32).reshape(1, 1, b)


def _argmax_tc(z, block_rows=32768):
    n, k = z.shape
    g = n // block_rows
    return pl.pallas_call(
        _argmax_body,
        grid=(g,),
        in_specs=[pl.BlockSpec((block_rows, k), lambda j: (j, 0))],
        out_specs=pl.BlockSpec((1, 1, block_rows), lambda j: (j, 0, 0)),
        out_shape=jax.ShapeDtypeStruct((g, 1, block_rows), jnp.int32),
    )(z)


def _gather_sc(table, idx1d, n, k):
    info = plsc.get_sparse_core_info()
    nc, ns = info.num_cores, info.num_subcores
    nw = nc * ns
    rows_per_w = n // nw
    chunk = 256
    n_chunks = rows_per_w // chunk
    gpc = chunk // 128
    mesh = plsc.VectorSubcoreMesh(core_axis_name="c", subcore_axis_name="s")

    nbuf = 3
    scratch = [
        pltpu.VMEM((rows_per_w,), jnp.int32),
        pltpu.VMEM((nbuf, chunk, k), jnp.float32),
        pltpu.VMEM_SHARED((k + 1, k), jnp.float32),
        pltpu.SemaphoreType.DMA,
    ] + [pltpu.SemaphoreType.DMA] * nbuf

    @functools.partial(
        pl.kernel,
        mesh=mesh,
        out_type=jax.ShapeDtypeStruct((n, k), jnp.float32),
        scratch_types=scratch,
    )
    def gather_kernel(table_hbm, idx_hbm, out_hbm, idx_v, rows_v, table_spm,
                      gsem, *osems):
        wid = lax.axis_index("s") * nc + lax.axis_index("c")
        row0 = wid * rows_per_w

        @pl.when(lax.axis_index("s") == 0)
        def _():
            pltpu.sync_copy(table_hbm, table_spm)
        pltpu.sync_copy(idx_hbm.at[pl.ds(row0, rows_per_w)], idx_v)
        plsc.subcore_barrier()

        def do_chunk(c, slot):
            off = row0 + c * chunk
            @pl.when(c >= nbuf)
            def _():
                pltpu.make_async_copy(
                    rows_v.at[slot],
                    out_hbm.at[pl.ds(off - nbuf * chunk, chunk)],
                    osems[slot],
                ).wait()
            handles = []
            for j in range(gpc):
                handles.append(pltpu.async_copy(
                    table_spm.at[idx_v.at[pl.ds(c * chunk + j * 128, 128)]],
                    rows_v.at[slot].at[pl.ds(j * 128, 128)],
                    gsem,
                ))
            for h in handles:
                h.wait()
            pltpu.async_copy(rows_v.at[slot],
                             out_hbm.at[pl.ds(off, chunk)], osems[slot])

        n_main = (n_chunks // nbuf) * nbuf

        def body(i2, carry):
            for b in range(nbuf):
                do_chunk(i2 * nbuf + b, b)
            return carry

        lax.fori_loop(0, n_chunks // nbuf, body, 0)
        for c in range(n_main, n_chunks):
            do_chunk(c, c % nbuf)
        for c in range(n_chunks - nbuf, n_chunks):
            slot = c % nbuf
            pltpu.make_async_copy(
                rows_v.at[slot],
                out_hbm.at[pl.ds(row0 + c * chunk, chunk)],
                osems[slot],
            ).wait()

    return gather_kernel(table, idx1d)


def kernel(z, A, z0):
    n, k = z.shape
    sel = _argmax_tc(z).reshape(n)
    idx = jnp.concatenate([jnp.full((1,), k, jnp.int32), sel[:-1]])
    table = jnp.concatenate([A, z0.astype(A.dtype)], axis=0)
    return _gather_sc(table, idx, n, k)


# speedup vs baseline: 4.0809x; 1.0031x over previous
import functools

import jax
import jax.numpy as jnp
from jax import lax
from jax.experimental import pallas as pl
from jax.experimental.pallas import tpu as pltpu
from jax.experimental.pallas import tpu_sc as plsc


def _argmax_body(z_ref, sel_ref):
    z = z_ref[...]
    b, k = z.shape
    m = jnp.max(z, axis=1, keepdims=True)
    eq = (z == m).astype(jnp.float32)
    iota_r = lax.broadcasted_iota(jnp.int32, (k, k), 0)
    iota_c = lax.broadcasted_iota(jnp.int32, (k, k), 1)
    ut = (iota_r <= iota_c).astype(jnp.float32)
    s = lax.dot_general(eq, ut, (((1,), (0,)), ((), ())),
                        preferred_element_type=jnp.float32)
    zeros = (s == 0).astype(jnp.float32)
    ones_row = jnp.ones((1, k), jnp.float32)
    selrow = lax.dot_general(ones_row, zeros, (((1,), (1,)), ((), ())),
                             preferred_element_type=jnp.float32)
    sel_ref[...] = selrow.astype(jnp.int32).reshape(1, 1, b)


def _argmax_tc(z, block_rows=40960):
    n, k = z.shape
    g = n // block_rows
    return pl.pallas_call(
        _argmax_body,
        grid=(g,),
        in_specs=[pl.BlockSpec((block_rows, k), lambda j: (j, 0))],
        out_specs=pl.BlockSpec((1, 1, block_rows), lambda j: (j, 0, 0)),
        out_shape=jax.ShapeDtypeStruct((g, 1, block_rows), jnp.int32),
    )(z)


def _gather_sc(table, idx1d, n, k):
    info = plsc.get_sparse_core_info()
    nc, ns = info.num_cores, info.num_subcores
    nw = nc * ns
    rows_per_w = n // nw
    chunk = 256
    n_chunks = rows_per_w // chunk
    gpc = chunk // 128
    mesh = plsc.VectorSubcoreMesh(core_axis_name="c", subcore_axis_name="s")

    nbuf = 3
    scratch = [
        pltpu.VMEM((rows_per_w,), jnp.int32),
        pltpu.VMEM((nbuf, chunk, k), jnp.float32),
        pltpu.VMEM_SHARED((k + 1, k), jnp.float32),
        pltpu.SemaphoreType.DMA,
    ] + [pltpu.SemaphoreType.DMA] * nbuf

    @functools.partial(
        pl.kernel,
        mesh=mesh,
        out_type=jax.ShapeDtypeStruct((n, k), jnp.float32),
        scratch_types=scratch,
    )
    def gather_kernel(table_hbm, idx_hbm, out_hbm, idx_v, rows_v, table_spm,
                      gsem, *osems):
        wid = lax.axis_index("s") * nc + lax.axis_index("c")
        row0 = wid * rows_per_w

        @pl.when(lax.axis_index("s") == 0)
        def _():
            pltpu.sync_copy(table_hbm, table_spm)
        pltpu.sync_copy(idx_hbm.at[pl.ds(row0, rows_per_w)], idx_v)
        plsc.subcore_barrier()

        def do_chunk(c, slot):
            off = row0 + c * chunk
            @pl.when(c >= nbuf)
            def _():
                pltpu.make_async_copy(
                    rows_v.at[slot],
                    out_hbm.at[pl.ds(off - nbuf * chunk, chunk)],
                    osems[slot],
                ).wait()
            handles = []
            for j in range(gpc):
                handles.append(pltpu.async_copy(
                    table_spm.at[idx_v.at[pl.ds(c * chunk + j * 128, 128)]],
                    rows_v.at[slot].at[pl.ds(j * 128, 128)],
                    gsem,
                ))
            for h in handles:
                h.wait()
            pltpu.async_copy(rows_v.at[slot],
                             out_hbm.at[pl.ds(off, chunk)], osems[slot])

        n_main = (n_chunks // nbuf) * nbuf

        def body(i2, carry):
            for b in range(nbuf):
                do_chunk(i2 * nbuf + b, b)
            return carry

        lax.fori_loop(0, n_chunks // nbuf, body, 0)
        for c in range(n_main, n_chunks):
            do_chunk(c, c % nbuf)
        for c in range(n_chunks - nbuf, n_chunks):
            slot = c % nbuf
            pltpu.make_async_copy(
                rows_v.at[slot],
                out_hbm.at[pl.ds(row0 + c * chunk, chunk)],
                osems[slot],
            ).wait()

    return gather_kernel(table, idx1d)


def kernel(z, A, z0):
    n, k = z.shape
    sel = _argmax_tc(z).reshape(n)
    idx = jnp.concatenate([jnp.full((1,), k, jnp.int32), sel[:-1]])
    table = jnp.concatenate([A, z0.astype(A.dtype)], axis=0)
    return _gather_sc(table, idx, n, k)
